# baseline (speedup 1.0000x reference)
import jax
import jax.numpy as jnp
from jax.experimental import pallas as pl
from jax.experimental.pallas import tpu as pltpu

D_MODEL_ = 2048
N_EXP_ = 64
BM_ = 1024


def _router_body(x_ref, wt_ref, b_ref, o_ref):
    o_ref[...] = (
        jnp.dot(x_ref[...], wt_ref[...], preferred_element_type=jnp.float32)
        + b_ref[...]
    )


def kernel(x, W, b):
    bsz, seq, d = x.shape
    m = bsz * seq
    x2 = x.reshape(m, d)
    wt = W.T
    b2 = b.reshape(1, N_EXP_)
    grid = (m // BM_,)
    out = pl.pallas_call(
        _router_body,
        grid=grid,
        in_specs=[
            pl.BlockSpec((BM_, d), lambda i: (i, 0)),
            pl.BlockSpec((d, N_EXP_), lambda i: (0, 0)),
            pl.BlockSpec((1, N_EXP_), lambda i: (0, 0)),
        ],
        out_specs=pl.BlockSpec((BM_, N_EXP_), lambda i: (i, 0)),
        out_shape=jax.ShapeDtypeStruct((m, N_EXP_), jnp.float32),
        compiler_params=pltpu.CompilerParams(
            dimension_semantics=("arbitrary",),
        ),
    )(x2, wt, b2)
    return out.reshape(bsz, seq, N_EXP_)
